# baseline (speedup 1.0000x reference)
import functools

import jax
import jax.numpy as jnp
from jax.experimental import pallas as pl
from jax.experimental.pallas import tpu as pltpu

_H = 32
_W = 32
_HW = _H * _W


def _conv_taps(x_pad, w_ref, taps=9):
    acc = None
    for t in range(taps):
        ky, kx = divmod(t, 3)
        xs = x_pad[ky:ky + _H, kx:kx + _W, :].reshape(_HW, x_pad.shape[-1])
        d = jnp.dot(xs, w_ref[t], preferred_element_type=jnp.float32)
        acc = d if acc is None else acc + d
    return acc


def _body(x_ref, w1_ref, b1_ref, w2_ref, b2_ref, w3_ref, b3_ref, o_ref,
          y1_ref, y2_ref):
    b = pl.program_id(0)

    @pl.when(b == 0)
    def _zero():
        y1_ref[...] = jnp.zeros_like(y1_ref)
        y2_ref[...] = jnp.zeros_like(y2_ref)

    cdt = y1_ref.dtype

    a1 = _conv_taps(x_ref[0], w1_ref)
    y1 = jnp.maximum(a1 + b1_ref[0], 0.0).astype(cdt)
    y1_ref[1:1 + _H, 1:1 + _W, :] = y1.reshape(_H, _W, y1.shape[-1])

    a2 = _conv_taps(y1_ref[...], w2_ref)
    y2 = jnp.maximum(a2 + b2_ref[0], 0.0).astype(cdt)
    y2_ref[1:1 + _H, 1:1 + _W, :] = y2.reshape(_H, _W, y2.shape[-1])

    a3 = _conv_taps(y2_ref[...], w3_ref)
    o_ref[0] = jax.nn.sigmoid(a3 + b3_ref[0])


@functools.partial(jax.jit, static_argnames=("cdt",))
def _saliency(x_pad, w1, b1, w2, b2, w3, b3, cdt=jnp.float32):
    B = x_pad.shape[0]
    hid = w1.shape[-1]
    grid = (B,)
    out = pl.pallas_call(
        _body,
        grid=grid,
        in_specs=[
            pl.BlockSpec((1, _H + 2, _W + 2, x_pad.shape[-1]),
                         lambda b: (b, 0, 0, 0)),
            pl.BlockSpec(w1.shape, lambda b: (0, 0, 0)),
            pl.BlockSpec(b1.shape, lambda b: (0, 0)),
            pl.BlockSpec(w2.shape, lambda b: (0, 0, 0)),
            pl.BlockSpec(b2.shape, lambda b: (0, 0)),
            pl.BlockSpec(w3.shape, lambda b: (0, 0, 0)),
            pl.BlockSpec(b3.shape, lambda b: (0, 0)),
        ],
        out_specs=pl.BlockSpec((1, _HW, 8), lambda b: (b, 0, 0)),
        out_shape=jax.ShapeDtypeStruct((B, _HW, 8), jnp.float32),
        scratch_shapes=[
            pltpu.VMEM((_H + 2, _W + 2, hid), cdt),
            pltpu.VMEM((_H + 2, _W + 2, hid), cdt),
        ],
        compiler_params=pltpu.CompilerParams(
            dimension_semantics=("arbitrary",),
        ),
    )(x_pad, w1, b1, w2, b2, w3, b3)
    return out


def _fold_bn(w, b, g, be, rm, rv, eps=1e-5):
    inv = g * jax.lax.rsqrt(rv + eps)
    return w * inv[:, None, None, None], (b - rm) * inv + be


def _tap_matrices(w):
    return jnp.transpose(w, (2, 3, 1, 0)).reshape(9, w.shape[1], w.shape[0])


_CDT = jnp.float32


def kernel(dino_features, W1, b1, g1, be1, rm1, rv1, W2, b2, g2, be2, rm2,
           rv2, W3, b3):
    B, H, W, C = dino_features.shape
    w1f, b1f = _fold_bn(W1, b1, g1, be1, rm1, rv1)
    w2f, b2f = _fold_bn(W2, b2, g2, be2, rm2, rv2)
    w1m = _tap_matrices(w1f).astype(_CDT)
    w2m = _tap_matrices(w2f).astype(_CDT)
    w3m = _tap_matrices(W3)
    w3m = jnp.pad(w3m, ((0, 0), (0, 0), (0, 7))).astype(_CDT)
    b3p = jnp.broadcast_to(b3, (8,)).reshape(1, 8).astype(jnp.float32)

    x_pad = jnp.pad(dino_features, ((0, 0), (1, 1), (1, 1), (0, 0)))
    x_pad = x_pad.astype(_CDT)

    out = _saliency(x_pad, w1m, b1f.reshape(1, -1).astype(jnp.float32),
                    w2m, b2f.reshape(1, -1).astype(jnp.float32),
                    w3m, b3p, cdt=_CDT)
    return out[:, :, 0].reshape(B, H, W, 1)


# speedup vs baseline: 1.0110x; 1.0110x over previous
import functools

import jax
import jax.numpy as jnp
from jax.experimental import pallas as pl
from jax.experimental.pallas import tpu as pltpu

_H = 32
_W = 32
_HW = _H * _W


def _conv_taps(x_pad, w_ref, taps=9):
    acc = None
    for t in range(taps):
        ky, kx = divmod(t, 3)
        xs = x_pad[ky:ky + _H, kx:kx + _W, :].reshape(_HW, x_pad.shape[-1])
        d = jnp.dot(xs, w_ref[t], preferred_element_type=jnp.float32)
        acc = d if acc is None else acc + d
    return acc


def _body(x_ref, w1_ref, b1_ref, w2_ref, b2_ref, w3_ref, b3_ref, o_ref,
          y1_ref, y2_ref):
    b = pl.program_id(0)

    @pl.when(b == 0)
    def _zero():
        y1_ref[...] = jnp.zeros_like(y1_ref)
        y2_ref[...] = jnp.zeros_like(y2_ref)

    cdt = y1_ref.dtype

    a1 = _conv_taps(x_ref[0], w1_ref)
    y1 = jnp.maximum(a1 + b1_ref[0], 0.0).astype(cdt)
    y1_ref[1:1 + _H, 1:1 + _W, :] = y1.reshape(_H, _W, y1.shape[-1])

    a2 = _conv_taps(y1_ref[...], w2_ref)
    y2 = jnp.maximum(a2 + b2_ref[0], 0.0).astype(cdt)
    y2_ref[1:1 + _H, 1:1 + _W, :] = y2.reshape(_H, _W, y2.shape[-1])

    a3 = _conv_taps(y2_ref[...], w3_ref)
    o_ref[0] = jax.nn.sigmoid(a3 + b3_ref[0])


@functools.partial(jax.jit, static_argnames=("cdt",))
def _saliency(x_pad, w1, b1, w2, b2, w3, b3, cdt=jnp.float32):
    B = x_pad.shape[0]
    hid = w1.shape[-1]
    grid = (B,)
    out = pl.pallas_call(
        _body,
        grid=grid,
        in_specs=[
            pl.BlockSpec((1, _H + 2, _W + 2, x_pad.shape[-1]),
                         lambda b: (b, 0, 0, 0)),
            pl.BlockSpec(w1.shape, lambda b: (0, 0, 0)),
            pl.BlockSpec(b1.shape, lambda b: (0, 0)),
            pl.BlockSpec(w2.shape, lambda b: (0, 0, 0)),
            pl.BlockSpec(b2.shape, lambda b: (0, 0)),
            pl.BlockSpec(w3.shape, lambda b: (0, 0, 0)),
            pl.BlockSpec(b3.shape, lambda b: (0, 0)),
        ],
        out_specs=pl.BlockSpec((1, _HW, 8), lambda b: (b, 0, 0)),
        out_shape=jax.ShapeDtypeStruct((B, _HW, 8), jnp.float32),
        scratch_shapes=[
            pltpu.VMEM((_H + 2, _W + 2, hid), cdt),
            pltpu.VMEM((_H + 2, _W + 2, hid), cdt),
        ],
        compiler_params=pltpu.CompilerParams(
            dimension_semantics=("arbitrary",),
        ),
    )(x_pad, w1, b1, w2, b2, w3, b3)
    return out


def _fold_bn(w, b, g, be, rm, rv, eps=1e-5):
    inv = g * jax.lax.rsqrt(rv + eps)
    return w * inv[:, None, None, None], (b - rm) * inv + be


def _tap_matrices(w):
    return jnp.transpose(w, (2, 3, 1, 0)).reshape(9, w.shape[1], w.shape[0])


_CDT = jnp.bfloat16


def kernel(dino_features, W1, b1, g1, be1, rm1, rv1, W2, b2, g2, be2, rm2,
           rv2, W3, b3):
    B, H, W, C = dino_features.shape
    w1f, b1f = _fold_bn(W1, b1, g1, be1, rm1, rv1)
    w2f, b2f = _fold_bn(W2, b2, g2, be2, rm2, rv2)
    w1m = _tap_matrices(w1f).astype(_CDT)
    w2m = _tap_matrices(w2f).astype(_CDT)
    w3m = _tap_matrices(W3)
    w3m = jnp.pad(w3m, ((0, 0), (0, 0), (0, 7))).astype(_CDT)
    b3p = jnp.broadcast_to(b3, (8,)).reshape(1, 8).astype(jnp.float32)

    x_pad = jnp.pad(dino_features, ((0, 0), (1, 1), (1, 1), (0, 0)))
    x_pad = x_pad.astype(_CDT)

    out = _saliency(x_pad, w1m, b1f.reshape(1, -1).astype(jnp.float32),
                    w2m, b2f.reshape(1, -1).astype(jnp.float32),
                    w3m, b3p, cdt=_CDT)
    return out[:, :, 0].reshape(B, H, W, 1)


# speedup vs baseline: 1.2973x; 1.2832x over previous
import functools

import jax
import jax.numpy as jnp
from jax.experimental import pallas as pl
from jax.experimental.pallas import tpu as pltpu

_H = 32
_W = 32
_HW = _H * _W
_ROWS = (_H + 2) * _W


def _stage_shifts(val, sc_ref):
    sc_ref[0, 1:1 + _H, 1:_W, :] = val[:, 0:_W - 1, :]
    sc_ref[1, 1:1 + _H, :, :] = val
    sc_ref[2, 1:1 + _H, 0:_W - 1, :] = val[:, 1:_W, :]


def _conv_from_shifts(sc_ref, w_ref, cout):
    cin = sc_ref.shape[-1]
    acc = None
    for kx in range(3):
        p = jnp.dot(sc_ref[kx].reshape(_ROWS, cin), w_ref[kx],
                    preferred_element_type=jnp.float32)
        for ky in range(3):
            term = p[_W * ky:_W * ky + _HW, cout * ky:cout * (ky + 1)]
            acc = term if acc is None else acc + term
    return acc


def _body(x_ref, w1_ref, s1_ref, b1_ref, w2_ref, s2_ref, b2_ref, w3_ref,
          b3_ref, o_ref, xs_ref, y1_ref, y2_ref):
    b = pl.program_id(0)

    @pl.when(b == 0)
    def _zero():
        xs_ref[...] = jnp.zeros_like(xs_ref)
        y1_ref[...] = jnp.zeros_like(y1_ref)
        y2_ref[...] = jnp.zeros_like(y2_ref)

    _stage_shifts(x_ref[0].astype(jnp.bfloat16), xs_ref)

    a1 = _conv_from_shifts(xs_ref, w1_ref, 256)
    y1 = jnp.maximum(a1 * s1_ref[0] + b1_ref[0], 0.0).astype(jnp.bfloat16)
    _stage_shifts(y1.reshape(_H, _W, -1), y1_ref)

    a2 = _conv_from_shifts(y1_ref, w2_ref, 256)
    y2 = jnp.maximum(a2 * s2_ref[0] + b2_ref[0], 0.0).astype(jnp.bfloat16)
    _stage_shifts(y2.reshape(_H, _W, -1), y2_ref)

    a3 = _conv_from_shifts(y2_ref, w3_ref, 1)
    out = jax.nn.sigmoid(a3 + b3_ref[0, 0:1])
    o_ref[0] = out


@jax.jit
def _saliency(x, w1, s1, b1, w2, s2, b2, w3, b3):
    B = x.shape[0]
    cin = x.shape[-1]
    hid = w1.shape[2] // 3
    out = pl.pallas_call(
        _body,
        grid=(B,),
        in_specs=[
            pl.BlockSpec((1, _H, _W, cin), lambda b: (b, 0, 0, 0)),
            pl.BlockSpec(w1.shape, lambda b: (0, 0, 0)),
            pl.BlockSpec(s1.shape, lambda b: (0, 0)),
            pl.BlockSpec(b1.shape, lambda b: (0, 0)),
            pl.BlockSpec(w2.shape, lambda b: (0, 0, 0)),
            pl.BlockSpec(s2.shape, lambda b: (0, 0)),
            pl.BlockSpec(b2.shape, lambda b: (0, 0)),
            pl.BlockSpec(w3.shape, lambda b: (0, 0, 0)),
            pl.BlockSpec(b3.shape, lambda b: (0, 0)),
        ],
        out_specs=pl.BlockSpec((1, _HW, 1), lambda b: (b, 0, 0)),
        out_shape=jax.ShapeDtypeStruct((B, _HW, 1), jnp.float32),
        scratch_shapes=[
            pltpu.VMEM((3, _H + 2, _W, cin), jnp.bfloat16),
            pltpu.VMEM((3, _H + 2, _W, hid), jnp.bfloat16),
            pltpu.VMEM((3, _H + 2, _W, hid), jnp.bfloat16),
        ],
        compiler_params=pltpu.CompilerParams(
            dimension_semantics=("arbitrary",),
        ),
    )(x, w1, s1, b1, w2, s2, b2, w3, b3)
    return out


def _kx_matrices(w, pad_to=None):
    m = jnp.transpose(w, (3, 1, 2, 0)).reshape(3, w.shape[1], 3 * w.shape[0])
    if pad_to is not None:
        m = jnp.pad(m, ((0, 0), (0, 0), (0, pad_to - m.shape[-1])))
    return m.astype(jnp.bfloat16)


def _bn_scale_bias(b, g, be, rm, rv, eps=1e-5):
    inv = g * jax.lax.rsqrt(rv + eps)
    return inv, (b - rm) * inv + be


def kernel(dino_features, W1, b1, g1, be1, rm1, rv1, W2, b2, g2, be2, rm2,
           rv2, W3, b3):
    B, H, W, C = dino_features.shape
    s1, b1f = _bn_scale_bias(b1, g1, be1, rm1, rv1)
    s2, b2f = _bn_scale_bias(b2, g2, be2, rm2, rv2)
    w1m = _kx_matrices(W1)
    w2m = _kx_matrices(W2)
    w3m = _kx_matrices(W3, pad_to=8)
    b3p = jnp.broadcast_to(b3, (8,)).reshape(1, 8).astype(jnp.float32)

    out = _saliency(dino_features,
                    w1m, s1.reshape(1, -1), b1f.reshape(1, -1),
                    w2m, s2.reshape(1, -1), b2f.reshape(1, -1),
                    w3m, b3p)
    return out.reshape(B, H, W, 1)


# speedup vs baseline: 1.8057x; 1.3919x over previous
import functools

import jax
import jax.numpy as jnp
from jax.experimental import pallas as pl
from jax.experimental.pallas import tpu as pltpu

_H = 32
_W = 32
_HW = _H * _W
_NB = 2


def _stage_shifts(val, sc_ref, bi):
    sc_ref[bi, 0, 1:1 + _H, 1:_W, :] = val[:, 0:_W - 1, :]
    sc_ref[bi, 1, 1:1 + _H, :, :] = val
    sc_ref[bi, 2, 1:1 + _H, 0:_W - 1, :] = val[:, 1:_W, :]


def _tree_sum(ts):
    while len(ts) > 1:
        ts = [a + b for a, b in zip(ts[::2], ts[1::2])] + \
            ([ts[-1]] if len(ts) % 2 else [])
    return ts[0]


def _conv_from_shifts(sc_ref, bi, w_ref):
    cin = sc_ref.shape[-1]
    terms = []
    for t in range(9):
        ky, kx = divmod(t, 3)
        lhs = sc_ref[bi, kx, ky:ky + _H, :, :].reshape(_HW, cin)
        terms.append(jnp.dot(lhs, w_ref[t],
                             preferred_element_type=jnp.float32))
    return _tree_sum(terms)


def _body(x_ref, w1_ref, s1_ref, b1_ref, w2_ref, s2_ref, b2_ref, w3_ref,
          b3_ref, o_ref, xs_ref, y1_ref, y2_ref):
    b = pl.program_id(0)

    @pl.when(b == 0)
    def _zero():
        xs_ref[...] = jnp.zeros_like(xs_ref)
        y1_ref[...] = jnp.zeros_like(y1_ref)
        y2_ref[...] = jnp.zeros_like(y2_ref)

    for bi in range(_NB):
        _stage_shifts(x_ref[bi].astype(jnp.bfloat16), xs_ref, bi)

    for bi in range(_NB):
        a1 = _conv_from_shifts(xs_ref, bi, w1_ref)
        y1 = jnp.maximum(a1 * s1_ref[0] + b1_ref[0], 0.0).astype(jnp.bfloat16)
        _stage_shifts(y1.reshape(_H, _W, -1), y1_ref, bi)

    for bi in range(_NB):
        a2 = _conv_from_shifts(y1_ref, bi, w2_ref)
        y2 = jnp.maximum(a2 * s2_ref[0] + b2_ref[0], 0.0).astype(jnp.bfloat16)
        _stage_shifts(y2.reshape(_H, _W, -1), y2_ref, bi)

    for bi in range(_NB):
        terms = []
        for kx in range(3):
            t = jnp.dot(y2_ref[bi, kx].reshape((_H + 2) * _W,
                                               y2_ref.shape[-1]),
                        w3_ref[kx], preferred_element_type=jnp.float32)
            t = t.reshape(_H + 2, _W, 8)
            for ky in range(3):
                terms.append(t[ky:ky + _H, :, ky:ky + 1])
        out = jax.nn.sigmoid(_tree_sum(terms) + b3_ref[0, 0:1])
        o_ref[bi] = out.reshape(_HW, 1)


@jax.jit
def _saliency(x, w1, s1, b1, w2, s2, b2, w3, b3):
    B = x.shape[0]
    cin = x.shape[-1]
    hid = w1.shape[-1]
    out = pl.pallas_call(
        _body,
        grid=(B // _NB,),
        in_specs=[
            pl.BlockSpec((_NB, _H, _W, cin), lambda b: (b, 0, 0, 0)),
            pl.BlockSpec(w1.shape, lambda b: (0, 0, 0)),
            pl.BlockSpec(s1.shape, lambda b: (0, 0)),
            pl.BlockSpec(b1.shape, lambda b: (0, 0)),
            pl.BlockSpec(w2.shape, lambda b: (0, 0, 0)),
            pl.BlockSpec(s2.shape, lambda b: (0, 0)),
            pl.BlockSpec(b2.shape, lambda b: (0, 0)),
            pl.BlockSpec(w3.shape, lambda b: (0, 0, 0)),
            pl.BlockSpec(b3.shape, lambda b: (0, 0)),
        ],
        out_specs=pl.BlockSpec((_NB, _HW, 1), lambda b: (b, 0, 0)),
        out_shape=jax.ShapeDtypeStruct((B, _HW, 1), jnp.float32),
        scratch_shapes=[
            pltpu.VMEM((_NB, 3, _H + 2, _W, cin), jnp.bfloat16),
            pltpu.VMEM((_NB, 3, _H + 2, _W, hid), jnp.bfloat16),
            pltpu.VMEM((_NB, 3, _H + 2, _W, hid), jnp.bfloat16),
        ],
        compiler_params=pltpu.CompilerParams(
            dimension_semantics=("arbitrary",),
        ),
    )(x, w1, s1, b1, w2, s2, b2, w3, b3)
    return out


def _tap_matrices(w):
    return jnp.transpose(w, (2, 3, 1, 0)).reshape(9, w.shape[1], w.shape[0])


def _bn_scale_bias(b, g, be, rm, rv, eps=1e-5):
    inv = g * jax.lax.rsqrt(rv + eps)
    return inv, (b - rm) * inv + be


def kernel(dino_features, W1, b1, g1, be1, rm1, rv1, W2, b2, g2, be2, rm2,
           rv2, W3, b3):
    B, H, W, C = dino_features.shape
    s1, b1f = _bn_scale_bias(b1, g1, be1, rm1, rv1)
    s2, b2f = _bn_scale_bias(b2, g2, be2, rm2, rv2)
    w1m = _tap_matrices(W1).astype(jnp.bfloat16)
    w2m = _tap_matrices(W2).astype(jnp.bfloat16)
    w3m = jnp.pad(jnp.transpose(W3[0], (2, 0, 1)), ((0, 0), (0, 0), (0, 5)))
    w3m = w3m.astype(jnp.bfloat16)
    b3p = jnp.broadcast_to(b3, (8,)).reshape(1, 8).astype(jnp.float32)

    out = _saliency(dino_features,
                    w1m, s1.reshape(1, -1), b1f.reshape(1, -1),
                    w2m, s2.reshape(1, -1), b2f.reshape(1, -1),
                    w3m, b3p)
    return out.reshape(B, H, W, 1)
